# Initial kernel scaffold; baseline (speedup 1.0000x reference)
import jax
import jax.numpy as jnp
from jax.experimental import pallas as pl


def kernel(x, params, edge_index, batch):
    raise NotImplementedError("write your pallas kernel here")


# speedup vs baseline: 3.9551x; 3.9551x over previous
import functools

import jax
import jax.numpy as jnp
from jax import lax
from jax.experimental import pallas as pl
from jax.experimental.pallas import tpu as pltpu
from jax.experimental.pallas import tpu_sc as plsc

N = 10000
E = 320000
FIN = 128
H = 256
G = 64
C = 10

NC = 2
NS = 16
K = 128
STAGE = 40
CCAP = 200
NROWS = 10240
ZSTRIPE = NROWS // NS


def _make_agg(fh):
    mesh = plsc.VectorSubcoreMesh(core_axis_name="c", subcore_axis_name="s")

    @functools.partial(
        pl.kernel,
        out_type=jax.ShapeDtypeStruct((NC, NROWS, fh), jnp.float32),
        mesh=mesh,
        scratch_types=[
            pltpu.VMEM((STAGE, K), jnp.int32),
            pltpu.VMEM((STAGE, K), jnp.int32),
            pltpu.VMEM((K, fh), jnp.float32),
            pltpu.VMEM((16,), jnp.int32),
            pltpu.VMEM_SHARED((NROWS, fh), jnp.float32),
            pltpu.SemaphoreType.DMA,
        ],
    )
    def agg_kernel(h2, srcoff, dstix, counts, zrows, out,
                   src_v, dst_v, rows_v, cnt_v, acc_sh, sem):
        cid = lax.axis_index("c")
        sid = lax.axis_index("s")
        pltpu.sync_copy(zrows, acc_sh.at[pl.ds(sid * ZSTRIPE, ZSTRIPE)])
        pltpu.sync_copy(counts.at[sid], cnt_v)
        plsc.subcore_barrier()
        trip = cnt_v[...][0]
        ngrp = (trip + (STAGE - 1)) // STAGE

        def group(g, carry):
            pltpu.sync_copy(srcoff.at[cid, sid, pl.ds(g * STAGE, STAGE)], src_v)
            pltpu.sync_copy(dstix.at[sid, pl.ds(g * STAGE, STAGE)], dst_v)

            def chunk(j, c2):
                pltpu.async_copy(h2.at[src_v.at[j]], rows_v, sem).wait()
                pltpu.sync_copy(rows_v, acc_sh.at[dst_v.at[j]], add=True)
                return c2

            return lax.fori_loop(0, jnp.minimum(STAGE, trip - g * STAGE),
                                 chunk, carry, unroll=False)

        lax.fori_loop(0, ngrp, group, 0, unroll=False)
        plsc.subcore_barrier()
        pltpu.sync_copy(
            acc_sh.at[pl.ds(sid * ZSTRIPE, ZSTRIPE)],
            out.at[cid, pl.ds(sid * ZSTRIPE, ZSTRIPE)],
        )

    return agg_kernel


_agg128 = _make_agg(128)


def _dot16(a, b):
    return jnp.dot(a.astype(jnp.bfloat16), b.astype(jnp.bfloat16),
                   preferred_element_type=jnp.float32)


def _mlp_body(h_ref, agg_ref, w1_ref, b1_ref, w2_ref, b2_ref, out_ref,
              *, split_in, split_out, relu_out):
    if split_in:
        h = jnp.concatenate([h_ref[0], h_ref[1]], axis=1)
    else:
        h = h_ref[...]
    agg = jnp.concatenate([agg_ref[0], agg_ref[1]], axis=1)
    z = h + agg
    t = jnp.maximum(_dot16(z, w1_ref[...]) + b1_ref[...], 0.0)
    o = _dot16(t, w2_ref[...]) + b2_ref[...]
    if relu_out:
        o = jnp.maximum(o, 0.0)
    if split_out:
        half = o.shape[1] // 2
        out_ref[0] = o[:, :half]
        out_ref[1] = o[:, half:]
    else:
        out_ref[...] = o


_BM = 2000
_NBLK = N // _BM


def _mlp_first_body(h_ref, agg_ref, w1_ref, b1_ref, w2_ref, b2_ref, out_ref):
    z = h_ref[...] + agg_ref[0]
    t = jnp.maximum(_dot16(z, w1_ref[...]) + b1_ref[...], 0.0)
    o = _dot16(t, w2_ref[...]) + b2_ref[...]
    o = jnp.maximum(o, 0.0)
    half = o.shape[1] // 2
    out_ref[0] = o[:, :half]
    out_ref[1] = o[:, half:]


def _mlp_first(x, agg, w1, b1, w2, b2):
    return pl.pallas_call(
        _mlp_first_body,
        grid=(_NBLK,),
        in_specs=[
            pl.BlockSpec((_BM, FIN), lambda i: (i, 0)),
            pl.BlockSpec((1, _BM, FIN), lambda i: (0, i, 0)),
            pl.BlockSpec((FIN, H), lambda i: (0, 0)),
            pl.BlockSpec((1, H), lambda i: (0, 0)),
            pl.BlockSpec((H, H), lambda i: (0, 0)),
            pl.BlockSpec((1, H), lambda i: (0, 0)),
        ],
        out_specs=pl.BlockSpec((NC, _BM, H // 2), lambda i: (0, i, 0)),
        out_shape=jax.ShapeDtypeStruct((NC, N, H // 2), jnp.float32),
    )(x, agg, w1, b1, w2, b2)


def _mlp_mid(h, agg, w1, b1, w2, b2):
    body = functools.partial(_mlp_body, split_in=True, split_out=True, relu_out=True)
    return pl.pallas_call(
        body,
        grid=(_NBLK,),
        in_specs=[
            pl.BlockSpec((NC, _BM, H // 2), lambda i: (0, i, 0)),
            pl.BlockSpec((NC, _BM, H // 2), lambda i: (0, i, 0)),
            pl.BlockSpec((H, H), lambda i: (0, 0)),
            pl.BlockSpec((1, H), lambda i: (0, 0)),
            pl.BlockSpec((H, H), lambda i: (0, 0)),
            pl.BlockSpec((1, H), lambda i: (0, 0)),
        ],
        out_specs=pl.BlockSpec((NC, _BM, H // 2), lambda i: (0, i, 0)),
        out_shape=jax.ShapeDtypeStruct((NC, N, H // 2), jnp.float32),
    )(h, agg, w1, b1, w2, b2)


def _mlp_last_body(h_ref, agg_ref, w1_ref, b1_ref, w2_ref, b2_ref,
                   out_ref, sums_ref, acc_ref):
    i = pl.program_id(0)
    h = jnp.concatenate([h_ref[0], h_ref[1]], axis=1)
    agg = jnp.concatenate([agg_ref[0], agg_ref[1]], axis=1)
    z = h + agg
    t = jnp.maximum(_dot16(z, w1_ref[...]) + b1_ref[...], 0.0)
    o = _dot16(t, w2_ref[...]) + b2_ref[...]
    out_ref[...] = o

    @pl.when(i == 0)
    def _():
        acc_ref[...] = jnp.zeros_like(acc_ref)

    acc_ref[0:1, :] += jnp.sum(o, axis=0, keepdims=True)
    acc_ref[1:2, :] += jnp.sum(o * o, axis=0, keepdims=True)

    @pl.when(i == _NBLK - 1)
    def _():
        sums_ref[...] = acc_ref[...]


def _mlp_last(h, agg, w1, b1, w2, b2):
    return pl.pallas_call(
        _mlp_last_body,
        grid=(_NBLK,),
        in_specs=[
            pl.BlockSpec((NC, _BM, H // 2), lambda i: (0, i, 0)),
            pl.BlockSpec((NC, _BM, H // 2), lambda i: (0, i, 0)),
            pl.BlockSpec((H, H), lambda i: (0, 0)),
            pl.BlockSpec((1, H), lambda i: (0, 0)),
            pl.BlockSpec((H, H), lambda i: (0, 0)),
            pl.BlockSpec((1, H), lambda i: (0, 0)),
        ],
        out_specs=[
            pl.BlockSpec((_BM, H), lambda i: (i, 0)),
            pl.BlockSpec((8, H), lambda i: (0, 0)),
        ],
        out_shape=[
            jax.ShapeDtypeStruct((N, H), jnp.float32),
            jax.ShapeDtypeStruct((8, H), jnp.float32),
        ],
        scratch_shapes=[pltpu.VMEM((8, H), jnp.float32)],
    )(h, agg, w1, b1, w2, b2)


_PBM = 1000
_PNBLK = N // _PBM


def _pool_body(h_ref, sums_ref, gam_ref, bet_ref, batr_ref, batc_ref, lohi_ref,
               f1w_ref, f1b_ref, f2w_ref, f2b_ref, out_ref,
               sc_ref, gmax_ref, gsum_ref, cnt_ref):
    i = pl.program_id(0)

    @pl.when(i == 0)
    def _():
        mean = sums_ref[0:1, :] * (1.0 / N)
        var = sums_ref[1:2, :] * (1.0 / N) - mean * mean
        inv = lax.rsqrt(var + 1e-5)
        scale = gam_ref[...] * inv
        sc_ref[0:1, :] = scale
        sc_ref[1:2, :] = bet_ref[...] - mean * scale
        gmax_ref[...] = jnp.full_like(gmax_ref, -jnp.inf)
        gsum_ref[...] = jnp.zeros_like(gsum_ref)
        cnt_ref[...] = jnp.zeros_like(cnt_ref)

    hn = h_ref[...] * sc_ref[0:1, :] + sc_ref[1:2, :]

    b_row = batr_ref[0]
    iota_g = lax.broadcasted_iota(jnp.int32, (G, _PBM), 0)
    oh = (b_row == iota_g).astype(jnp.float32)
    gsum_ref[...] += jnp.dot(oh, hn, preferred_element_type=jnp.float32,
                             precision=lax.Precision.HIGHEST)
    cnt_ref[...] += jnp.broadcast_to(jnp.sum(oh, axis=1, keepdims=True), cnt_ref.shape)

    b_col = batc_ref[0]
    g_lo = lohi_ref[i, 0]
    g_hi = lohi_ref[i, 1]

    def gbody(g, carry):
        m = jnp.where(b_col == g, 0.0, -jnp.inf)
        colmax = jnp.max(hn + m, axis=0, keepdims=True)
        gmax_ref[pl.ds(g, 1), :] = jnp.maximum(gmax_ref[pl.ds(g, 1), :], colmax)
        return carry

    lax.fori_loop(g_lo, g_hi + 1, gbody, 0)

    @pl.when(i == _PNBLK - 1)
    def _():
        cnt = cnt_ref[:, 0:1]
        gmean = gsum_ref[...] / jnp.maximum(cnt, 1.0)
        rep = jnp.concatenate([gmax_ref[...], gmean], axis=1)
        u = jnp.maximum(_dot16(rep, f1w_ref[...]) + f1b_ref[...], 0.0)
        out_ref[...] = _dot16(u, f2w_ref[...]) + f2b_ref[...]


def _pool(h, sums, gamma, beta, batr, batc, lohi, f1w, f1b, f2w, f2b):
    return pl.pallas_call(
        _pool_body,
        grid=(_PNBLK,),
        in_specs=[
            pl.BlockSpec((_PBM, H), lambda i: (i, 0)),
            pl.BlockSpec((8, H), lambda i: (0, 0)),
            pl.BlockSpec((1, H), lambda i: (0, 0)),
            pl.BlockSpec((1, H), lambda i: (0, 0)),
            pl.BlockSpec((1, 1, _PBM), lambda i: (i, 0, 0)),
            pl.BlockSpec((1, _PBM, 1), lambda i: (i, 0, 0)),
            pl.BlockSpec(memory_space=pltpu.SMEM),
            pl.BlockSpec((2 * H, H), lambda i: (0, 0)),
            pl.BlockSpec((1, H), lambda i: (0, 0)),
            pl.BlockSpec((H, C), lambda i: (0, 0)),
            pl.BlockSpec((1, C), lambda i: (0, 0)),
        ],
        out_specs=pl.BlockSpec((G, C), lambda i: (0, 0)),
        out_shape=jax.ShapeDtypeStruct((G, C), jnp.float32),
        scratch_shapes=[
            pltpu.VMEM((8, H), jnp.float32),
            pltpu.VMEM((G, H), jnp.float32),
            pltpu.VMEM((G, H), jnp.float32),
            pltpu.VMEM((G, 128), jnp.float32),
        ],
    )(h, sums, gamma, beta, batr, batc, lohi, f1w, f1b, f2w, f2b)


def kernel(x, params, edge_index, batch):
    src = edge_index[0].astype(jnp.int32)
    dst = edge_index[1].astype(jnp.int32)
    perm = jnp.argsort(dst, stable=True)
    srcs = src[perm]
    dsts = dst[perm]
    bounds = jnp.searchsorted(dsts, jnp.arange(NS + 1, dtype=jnp.int32) * ZSTRIPE
                              ).astype(jnp.int32)
    n_t = bounds[1:] - bounds[:-1]
    trips = (n_t + (K - 1)) // K
    counts = jnp.broadcast_to(trips[:, None], (NS, 16)).astype(jnp.int32)
    pos = jnp.arange(CCAP * K, dtype=jnp.int32)[None, :]
    idxs = bounds[:-1, None] + pos
    valid = pos < n_t[:, None]
    tile_src = jnp.where(valid, srcs[jnp.clip(idxs, 0, E - 1)], pos % N)
    tile_dst = jnp.where(valid, dsts[jnp.clip(idxs, 0, E - 1)],
                         N + (pos % (NROWS - N)))
    dstix = tile_dst.reshape(NS, CCAP, K)
    srcoff = (tile_src[None, :, :] +
              (jnp.arange(NC, dtype=jnp.int32) * N)[:, None, None]
              ).reshape(NC, NS, CCAP, K)
    zr128 = jnp.zeros((ZSTRIPE, 128), jnp.float32)

    gin = params["gin"]
    w = [(g["W1"], g["b1"].reshape(1, H), g["W2"], g["b2"].reshape(1, H))
         for g in gin]

    x2 = jnp.concatenate([x, x], axis=0)
    agg = _agg128(x2, srcoff, dstix, counts, zr128)
    h = _mlp_first(x, agg, *w[0])
    for l in (1, 2):
        agg = _agg128(h.reshape(NC * N, H // 2), srcoff, dstix, counts, zr128)
        h = _mlp_mid(h, agg, *w[l])
    agg = _agg128(h.reshape(NC * N, H // 2), srcoff, dstix, counts, zr128)
    h4, sums = _mlp_last(h, agg, *w[3])

    br = batch.astype(jnp.int32).reshape(_PNBLK, _PBM)
    lohi = jnp.stack([br[:, 0], br[:, -1]], axis=1)
    batr = br.reshape(_PNBLK, 1, _PBM)
    batc = br.reshape(_PNBLK, _PBM, 1)
    return _pool(
        h4, sums,
        params["bn_gamma"].reshape(1, H), params["bn_beta"].reshape(1, H),
        batr, batc, lohi,
        params["fc1_W"], params["fc1_b"].reshape(1, H),
        params["fc2_W"], params["fc2_b"].reshape(1, C),
    )
